# baseline (speedup 1.0000x reference)
import jax
import jax.numpy as jnp
from jax import lax
from jax.experimental import pallas as pl

_TOP_K = 8
_N_ROUTED = 64
_N_GROUP = 8
_TOPK_GROUP = 4
_SCALING = 2.5
_GROUP_SIZE = _N_ROUTED // _N_GROUP

_NEG_INF = float("-inf")


def _tree_max(vals):
    vals = list(vals)
    while len(vals) > 1:
        nxt = [jnp.maximum(vals[i], vals[i + 1])
               for i in range(0, len(vals) - 1, 2)]
        if len(vals) % 2:
            nxt.append(vals[-1])
        vals = nxt
    return vals[0]


def _tree_min(vals):
    vals = list(vals)
    while len(vals) > 1:
        nxt = [jnp.minimum(vals[i], vals[i + 1])
               for i in range(0, len(vals) - 1, 2)]
        if len(vals) % 2:
            nxt.append(vals[-1])
        vals = nxt
    return vals[0]


def _route_kernel(hs_ref, wt_ref, b_ref, idx_ref, wgt_ref):
    wt = wt_ref[...]
    b = b_ref[...]
    t = hs_ref.shape[0]
    chunk = 1024
    for c in range(t // chunk):
        hs = hs_ref[pl.ds(c * chunk, chunk), :]
        logits_t = lax.dot_general(
            wt, hs, (((1,), (1,)), ((), ())),
            preferred_element_type=jnp.float32)
        scores_t = jax.nn.sigmoid(logits_t) + b
        _route_chunk(scores_t, c, idx_ref, wgt_ref)


def _route_chunk(scores_t, c, idx_ref, wgt_ref):
    sub = scores_t.shape[1] // 128
    s3 = scores_t.reshape(_N_ROUTED, sub, 128)
    s = [s3[e] for e in range(_N_ROUTED)]

    shape = (sub, 128)
    neg = jnp.full(shape, _NEG_INF, jnp.float32)

    gval = []
    for g in range(_N_GROUP):
        m1 = s[g * _GROUP_SIZE]
        m2 = neg
        for j in range(1, _GROUP_SIZE):
            x = s[g * _GROUP_SIZE + j]
            lo = jnp.minimum(m1, x)
            m1 = jnp.maximum(m1, x)
            m2 = jnp.maximum(m2, lo)
        gval.append(m1 + m2)

    sel = [None] * _N_GROUP
    gw = list(gval)
    big_g = jnp.full(shape, _N_GROUP, jnp.int32)
    for r in range(_TOPK_GROUP):
        m = _tree_max(gw)
        widx = _tree_min(
            [jnp.where(gw[g] == m, g, big_g) for g in range(_N_GROUP)])
        for g in range(_N_GROUP):
            hit = widx == g
            sel[g] = hit if r == 0 else (sel[g] | hit)
            gw[g] = jnp.where(hit, neg, gw[g])

    ms = [jnp.where(sel[e // _GROUP_SIZE], s[e], 0.0)
          for e in range(_N_ROUTED)]

    wsum = jnp.zeros(shape, jnp.float32)
    big_e = jnp.full(shape, _N_ROUTED, jnp.int32)
    idxs = []
    wts = []
    for _ in range(_TOP_K):
        m = _tree_max(ms)
        widx = _tree_min(
            [jnp.where(ms[e] == m, e, big_e) for e in range(_N_ROUTED)])
        for e in range(_N_ROUTED):
            ms[e] = jnp.where(widx == e, neg, ms[e])
        idxs.append(widx)
        wts.append(m)
        wsum = wsum + m

    inv = _SCALING / (wsum + 1e-20)
    for k in range(_TOP_K):
        idx_ref[k, c] = idxs[k]
        wgt_ref[k, c] = wts[k] * inv


def kernel(hidden_states, kernel, e_score_correction_bias):
    bsz, seq_len, h = hidden_states.shape
    n = bsz * seq_len
    hs = hidden_states.reshape(n, h)
    wt = kernel.astype(jnp.float32).T
    b2d = e_score_correction_bias.reshape(_N_ROUTED, 1).astype(jnp.float32)

    block_t = 8192
    chunks_per_blk = block_t // 1024
    sub = 1024 // 128
    nchunk = n // 1024
    grid = (n // block_t,)
    out_shape = [
        jax.ShapeDtypeStruct((_TOP_K, nchunk, sub, 128), jnp.int32),
        jax.ShapeDtypeStruct((_TOP_K, nchunk, sub, 128), jnp.float32),
    ]
    idx4, wgt4 = pl.pallas_call(
        _route_kernel,
        grid=grid,
        in_specs=[
            pl.BlockSpec((block_t, h), lambda i: (i, 0)),
            pl.BlockSpec((_N_ROUTED, h), lambda i: (0, 0)),
            pl.BlockSpec((_N_ROUTED, 1), lambda i: (0, 0)),
        ],
        out_specs=[
            pl.BlockSpec((_TOP_K, chunks_per_blk, sub, 128),
                         lambda i: (0, i, 0, 0)),
            pl.BlockSpec((_TOP_K, chunks_per_blk, sub, 128),
                         lambda i: (0, i, 0, 0)),
        ],
        out_shape=out_shape,
    )(hs, wt, b2d)
    topk_idx = jnp.transpose(idx4, (1, 2, 3, 0)).reshape(n, _TOP_K)
    topk_wgt = jnp.transpose(wgt4, (1, 2, 3, 0)).reshape(n, _TOP_K)
    return (topk_idx, topk_wgt)


# speedup vs baseline: 1.0334x; 1.0334x over previous
import jax
import jax.numpy as jnp
from jax import lax
from jax.experimental import pallas as pl

_TOP_K = 8
_N_ROUTED = 64
_N_GROUP = 8
_TOPK_GROUP = 4
_SCALING = 2.5
_GROUP_SIZE = _N_ROUTED // _N_GROUP

_NEG_INF = float("-inf")


def _tree_max(vals):
    vals = list(vals)
    while len(vals) > 1:
        nxt = [jnp.maximum(vals[i], vals[i + 1])
               for i in range(0, len(vals) - 1, 2)]
        if len(vals) % 2:
            nxt.append(vals[-1])
        vals = nxt
    return vals[0]


def _tree_min(vals):
    vals = list(vals)
    while len(vals) > 1:
        nxt = [jnp.minimum(vals[i], vals[i + 1])
               for i in range(0, len(vals) - 1, 2)]
        if len(vals) % 2:
            nxt.append(vals[-1])
        vals = nxt
    return vals[0]


def _route_kernel(hs_ref, wt_ref, b_ref, idx_ref, wgt_ref):
    wt = wt_ref[...]
    b = b_ref[...]
    t = hs_ref.shape[0]
    chunk = 1024
    for c in range(t // chunk):
        hs = hs_ref[pl.ds(c * chunk, chunk), :]
        half = hs.shape[1] // 2
        logits_a = lax.dot_general(
            wt[:, :half], hs[:, :half], (((1,), (1,)), ((), ())),
            preferred_element_type=jnp.float32)
        logits_b = lax.dot_general(
            wt[:, half:], hs[:, half:], (((1,), (1,)), ((), ())),
            preferred_element_type=jnp.float32)
        scores_t = jax.nn.sigmoid(logits_a + logits_b) + b
        _route_chunk(scores_t, c, idx_ref, wgt_ref)


def _route_chunk(scores_t, c, idx_ref, wgt_ref):
    sub = scores_t.shape[1] // 128
    s3 = scores_t.reshape(_N_ROUTED, sub, 128)
    s = [s3[e] for e in range(_N_ROUTED)]

    shape = (sub, 128)
    neg = jnp.full(shape, _NEG_INF, jnp.float32)

    gval = []
    for g in range(_N_GROUP):
        m1 = s[g * _GROUP_SIZE]
        m2 = neg
        for j in range(1, _GROUP_SIZE):
            x = s[g * _GROUP_SIZE + j]
            lo = jnp.minimum(m1, x)
            m1 = jnp.maximum(m1, x)
            m2 = jnp.maximum(m2, lo)
        gval.append(m1 + m2)

    sel = [None] * _N_GROUP
    gw = list(gval)
    big_g = jnp.full(shape, _N_GROUP, jnp.int32)
    for r in range(_TOPK_GROUP):
        m = _tree_max(gw)
        widx = _tree_min(
            [jnp.where(gw[g] == m, g, big_g) for g in range(_N_GROUP)])
        for g in range(_N_GROUP):
            hit = widx == g
            sel[g] = hit if r == 0 else (sel[g] | hit)
            gw[g] = jnp.where(hit, neg, gw[g])

    ms = [jnp.where(sel[e // _GROUP_SIZE], s[e], 0.0)
          for e in range(_N_ROUTED)]

    wsum = jnp.zeros(shape, jnp.float32)
    big_e = jnp.full(shape, _N_ROUTED, jnp.int32)
    idxs = []
    wts = []
    for _ in range(_TOP_K):
        m = _tree_max(ms)
        widx = _tree_min(
            [jnp.where(ms[e] == m, e, big_e) for e in range(_N_ROUTED)])
        for e in range(_N_ROUTED):
            ms[e] = jnp.where(widx == e, neg, ms[e])
        idxs.append(widx)
        wts.append(m)
        wsum = wsum + m

    inv = _SCALING / (wsum + 1e-20)
    for k in range(_TOP_K):
        idx_ref[k, c] = idxs[k]
        wgt_ref[k, c] = wts[k] * inv


def kernel(hidden_states, kernel, e_score_correction_bias):
    bsz, seq_len, h = hidden_states.shape
    n = bsz * seq_len
    hs = hidden_states.reshape(n, h)
    wt = kernel.astype(jnp.float32).T
    b2d = e_score_correction_bias.reshape(_N_ROUTED, 1).astype(jnp.float32)

    block_t = 4096
    chunks_per_blk = block_t // 1024
    sub = 1024 // 128
    nchunk = n // 1024
    grid = (n // block_t,)
    out_shape = [
        jax.ShapeDtypeStruct((_TOP_K, nchunk, sub, 128), jnp.int32),
        jax.ShapeDtypeStruct((_TOP_K, nchunk, sub, 128), jnp.float32),
    ]
    idx4, wgt4 = pl.pallas_call(
        _route_kernel,
        grid=grid,
        in_specs=[
            pl.BlockSpec((block_t, h), lambda i: (i, 0)),
            pl.BlockSpec((_N_ROUTED, h), lambda i: (0, 0)),
            pl.BlockSpec((_N_ROUTED, 1), lambda i: (0, 0)),
        ],
        out_specs=[
            pl.BlockSpec((_TOP_K, chunks_per_blk, sub, 128),
                         lambda i: (0, i, 0, 0)),
            pl.BlockSpec((_TOP_K, chunks_per_blk, sub, 128),
                         lambda i: (0, i, 0, 0)),
        ],
        out_shape=out_shape,
    )(hs, wt, b2d)
    topk_idx = jnp.transpose(idx4, (1, 2, 3, 0)).reshape(n, _TOP_K)
    topk_wgt = jnp.transpose(wgt4, (1, 2, 3, 0)).reshape(n, _TOP_K)
    return (topk_idx, topk_wgt)


# speedup vs baseline: 1.0520x; 1.0180x over previous
import jax
import jax.numpy as jnp
from jax import lax
from jax.experimental import pallas as pl

_TOP_K = 8
_N_ROUTED = 64
_N_GROUP = 8
_TOPK_GROUP = 4
_SCALING = 2.5
_GROUP_SIZE = _N_ROUTED // _N_GROUP

_NEG_INF = float("-inf")


def _tree_max(vals):
    vals = list(vals)
    while len(vals) > 1:
        nxt = [jnp.maximum(vals[i], vals[i + 1])
               for i in range(0, len(vals) - 1, 2)]
        if len(vals) % 2:
            nxt.append(vals[-1])
        vals = nxt
    return vals[0]


def _tree_min(vals):
    vals = list(vals)
    while len(vals) > 1:
        nxt = [jnp.minimum(vals[i], vals[i + 1])
               for i in range(0, len(vals) - 1, 2)]
        if len(vals) % 2:
            nxt.append(vals[-1])
        vals = nxt
    return vals[0]


def _route_kernel(hs_ref, wt_ref, b_ref, idx_ref, wgt_ref):
    wt = wt_ref[...]
    b = b_ref[...]
    t = hs_ref.shape[0]
    chunk = 1024
    for c in range(t // chunk):
        hs = hs_ref[pl.ds(c * chunk, chunk), :]
        half = chunk // 2
        logits_a = lax.dot_general(
            wt, hs[:half], (((1,), (1,)), ((), ())),
            preferred_element_type=jnp.float32)
        logits_b = lax.dot_general(
            wt, hs[half:], (((1,), (1,)), ((), ())),
            preferred_element_type=jnp.float32)
        logits_t = jnp.concatenate([logits_a, logits_b], axis=1)
        scores_t = jax.nn.sigmoid(logits_t) + b
        _route_chunk(scores_t, c, idx_ref, wgt_ref)


def _route_chunk(scores_t, c, idx_ref, wgt_ref):
    sub = scores_t.shape[1] // 128
    s3 = scores_t.reshape(_N_ROUTED, sub, 128)
    s = [s3[e] for e in range(_N_ROUTED)]

    shape = (sub, 128)
    neg = jnp.full(shape, _NEG_INF, jnp.float32)

    gval = []
    for g in range(_N_GROUP):
        m1 = s[g * _GROUP_SIZE]
        m2 = neg
        for j in range(1, _GROUP_SIZE):
            x = s[g * _GROUP_SIZE + j]
            lo = jnp.minimum(m1, x)
            m1 = jnp.maximum(m1, x)
            m2 = jnp.maximum(m2, lo)
        gval.append(m1 + m2)

    sel = [None] * _N_GROUP
    gw = list(gval)
    big_g = jnp.full(shape, _N_GROUP, jnp.int32)
    for r in range(_TOPK_GROUP):
        m = _tree_max(gw)
        widx = _tree_min(
            [jnp.where(gw[g] == m, g, big_g) for g in range(_N_GROUP)])
        for g in range(_N_GROUP):
            hit = widx == g
            sel[g] = hit if r == 0 else (sel[g] | hit)
            gw[g] = jnp.where(hit, neg, gw[g])

    ms = [jnp.where(sel[e // _GROUP_SIZE], s[e], 0.0)
          for e in range(_N_ROUTED)]

    wsum = jnp.zeros(shape, jnp.float32)
    big_e = jnp.full(shape, _N_ROUTED, jnp.int32)
    idxs = []
    wts = []
    for _ in range(_TOP_K):
        m = _tree_max(ms)
        widx = _tree_min(
            [jnp.where(ms[e] == m, e, big_e) for e in range(_N_ROUTED)])
        for e in range(_N_ROUTED):
            ms[e] = jnp.where(widx == e, neg, ms[e])
        idxs.append(widx)
        wts.append(m)
        wsum = wsum + m

    inv = _SCALING / (wsum + 1e-20)
    for k in range(_TOP_K):
        idx_ref[k, c] = idxs[k]
        wgt_ref[k, c] = wts[k] * inv


def kernel(hidden_states, kernel, e_score_correction_bias):
    bsz, seq_len, h = hidden_states.shape
    n = bsz * seq_len
    hs = hidden_states.reshape(n, h)
    wt = kernel.astype(jnp.float32).T
    b2d = e_score_correction_bias.reshape(_N_ROUTED, 1).astype(jnp.float32)

    block_t = 4096
    chunks_per_blk = block_t // 1024
    sub = 1024 // 128
    nchunk = n // 1024
    grid = (n // block_t,)
    out_shape = [
        jax.ShapeDtypeStruct((_TOP_K, nchunk, sub, 128), jnp.int32),
        jax.ShapeDtypeStruct((_TOP_K, nchunk, sub, 128), jnp.float32),
    ]
    idx4, wgt4 = pl.pallas_call(
        _route_kernel,
        grid=grid,
        in_specs=[
            pl.BlockSpec((block_t, h), lambda i: (i, 0)),
            pl.BlockSpec((_N_ROUTED, h), lambda i: (0, 0)),
            pl.BlockSpec((_N_ROUTED, 1), lambda i: (0, 0)),
        ],
        out_specs=[
            pl.BlockSpec((_TOP_K, chunks_per_blk, sub, 128),
                         lambda i: (0, i, 0, 0)),
            pl.BlockSpec((_TOP_K, chunks_per_blk, sub, 128),
                         lambda i: (0, i, 0, 0)),
        ],
        out_shape=out_shape,
    )(hs, wt, b2d)
    topk_idx = jnp.transpose(idx4, (1, 2, 3, 0)).reshape(n, _TOP_K)
    topk_wgt = jnp.transpose(wgt4, (1, 2, 3, 0)).reshape(n, _TOP_K)
    return (topk_idx, topk_wgt)


# speedup vs baseline: 1.0584x; 1.0061x over previous
import jax
import jax.numpy as jnp
from jax import lax
from jax.experimental import pallas as pl

_TOP_K = 8
_N_ROUTED = 64
_N_GROUP = 8
_TOPK_GROUP = 4
_SCALING = 2.5
_GROUP_SIZE = _N_ROUTED // _N_GROUP

_NEG_INF = float("-inf")


def _tree_max(vals):
    vals = list(vals)
    while len(vals) > 1:
        nxt = [jnp.maximum(vals[i], vals[i + 1])
               for i in range(0, len(vals) - 1, 2)]
        if len(vals) % 2:
            nxt.append(vals[-1])
        vals = nxt
    return vals[0]


def _tree_min(vals):
    vals = list(vals)
    while len(vals) > 1:
        nxt = [jnp.minimum(vals[i], vals[i + 1])
               for i in range(0, len(vals) - 1, 2)]
        if len(vals) % 2:
            nxt.append(vals[-1])
        vals = nxt
    return vals[0]


def _route_kernel(hs_ref, wt_ref, b_ref, idx_ref, wgt_ref):
    wt = wt_ref[...]
    b = b_ref[...]
    t = hs_ref.shape[0]
    chunk = 1024
    for c in range(t // chunk):
        hs = hs_ref[pl.ds(c * chunk, chunk), :]
        logits_t = lax.dot_general(
            wt, hs, (((1,), (1,)), ((), ())),
            preferred_element_type=jnp.float32)
        scores_t = jax.nn.sigmoid(logits_t) + b
        _route_chunk(scores_t, c, idx_ref, wgt_ref)


def _route_chunk(scores_t, c, idx_ref, wgt_ref):
    sub = scores_t.shape[1] // 128
    s3 = scores_t.reshape(_N_ROUTED, sub, 128)
    s = [s3[e] for e in range(_N_ROUTED)]

    shape = (sub, 128)
    neg = jnp.full(shape, _NEG_INF, jnp.float32)

    gval = []
    for g in range(_N_GROUP):
        m1 = s[g * _GROUP_SIZE]
        m2 = neg
        for j in range(1, _GROUP_SIZE):
            x = s[g * _GROUP_SIZE + j]
            lo = jnp.minimum(m1, x)
            m1 = jnp.maximum(m1, x)
            m2 = jnp.maximum(m2, lo)
        gval.append(m1 + m2)

    sel = [None] * _N_GROUP
    gw = list(gval)
    big_g = jnp.full(shape, _N_GROUP, jnp.int32)
    for r in range(_TOPK_GROUP):
        m = _tree_max(gw)
        widx = _tree_min(
            [jnp.where(gw[g] == m, g, big_g) for g in range(_N_GROUP)])
        for g in range(_N_GROUP):
            hit = widx == g
            sel[g] = hit if r == 0 else (sel[g] | hit)
            gw[g] = jnp.where(hit, neg, gw[g])

    ms = [jnp.where(sel[e // _GROUP_SIZE], s[e], 0.0)
          for e in range(_N_ROUTED)]

    wsum = jnp.zeros(shape, jnp.float32)
    big_e = jnp.full(shape, _N_ROUTED, jnp.int32)
    idxs = []
    wts = []
    for _ in range(_TOP_K):
        m = _tree_max(ms)
        widx = _tree_min(
            [jnp.where(ms[e] == m, e, big_e) for e in range(_N_ROUTED)])
        for e in range(_N_ROUTED):
            ms[e] = jnp.where(widx == e, neg, ms[e])
        idxs.append(widx)
        wts.append(m)
        wsum = wsum + m

    inv = _SCALING / (wsum + 1e-20)
    for k in range(_TOP_K):
        idx_ref[k, c] = idxs[k]
        wgt_ref[k, c] = wts[k] * inv


def kernel(hidden_states, kernel, e_score_correction_bias):
    bsz, seq_len, h = hidden_states.shape
    n = bsz * seq_len
    hs = hidden_states.reshape(n, h)
    wt = kernel.astype(jnp.float32).T
    b2d = e_score_correction_bias.reshape(_N_ROUTED, 1).astype(jnp.float32)

    block_t = 4096
    chunks_per_blk = block_t // 1024
    sub = 1024 // 128
    nchunk = n // 1024
    grid = (n // block_t,)
    out_shape = [
        jax.ShapeDtypeStruct((_TOP_K, nchunk, sub, 128), jnp.int32),
        jax.ShapeDtypeStruct((_TOP_K, nchunk, sub, 128), jnp.float32),
    ]
    idx4, wgt4 = pl.pallas_call(
        _route_kernel,
        grid=grid,
        in_specs=[
            pl.BlockSpec((block_t, h), lambda i: (i, 0)),
            pl.BlockSpec((_N_ROUTED, h), lambda i: (0, 0)),
            pl.BlockSpec((_N_ROUTED, 1), lambda i: (0, 0)),
        ],
        out_specs=[
            pl.BlockSpec((_TOP_K, chunks_per_blk, sub, 128),
                         lambda i: (0, i, 0, 0)),
            pl.BlockSpec((_TOP_K, chunks_per_blk, sub, 128),
                         lambda i: (0, i, 0, 0)),
        ],
        out_shape=out_shape,
    )(hs, wt, b2d)
    topk_idx = jnp.transpose(idx4, (1, 2, 3, 0)).reshape(n, _TOP_K)
    topk_wgt = jnp.transpose(wgt4, (1, 2, 3, 0)).reshape(n, _TOP_K)
    return (topk_idx, topk_wgt)
